# baseline (speedup 1.0000x reference)
import functools

import jax
import jax.numpy as jnp
from jax import lax
from jax.experimental import pallas as pl
from jax.experimental.pallas import tpu as pltpu
from jax.experimental.pallas import tpu_sc as plsc

B, L, DIM = 1024, 200, 128
DAY_SIZE, TIME_SIZE = 32, 64
NUM_DAYS, DAILY_TIMES = 7, 288
DT = DAY_SIZE + TIME_SIZE
OUT_D = DIM + DT
N = B * L

_info = plsc.get_sparse_core_info()
NC, NS, LANES = _info.num_cores, _info.num_subcores, _info.num_lanes
NW = NC * NS
BTILES = B // 128
NUNIT = L * BTILES
UPW = NUNIT // NW
NBUF = 5
NOUTER = UPW // NBUF

_mesh = plsc.VectorSubcoreMesh(core_axis_name="c", subcore_axis_name="s")


@functools.partial(
    pl.kernel,
    out_type=jax.ShapeDtypeStruct((L, OUT_D, B), jnp.float32),
    mesh=_mesh,
    compiler_params=pltpu.CompilerParams(use_tc_tiling_on_sc=True,
                                         needs_layout_passes=False),
    scratch_types=(
        [pltpu.VMEM((2, 128), jnp.int32)] * NBUF
        + [pltpu.VMEM((DT, 128), jnp.float32)] * NBUF
        + [pltpu.VMEM((NUM_DAYS * DAY_SIZE,), jnp.float32)]
        + [pltpu.VMEM((DAILY_TIMES * TIME_SIZE,), jnp.float32)]
        + [pltpu.SemaphoreType.DMA] * (2 * NBUF)
    ),
)
def _sc_body(idx_hbm, day_hbm, time_hbm, out_hbm, *scratch):
    idx_v = scratch[0:NBUF]
    dt_v = scratch[NBUF:2 * NBUF]
    day_tab = scratch[2 * NBUF]
    time_tab = scratch[2 * NBUF + 1]
    in_sem = scratch[2 * NBUF + 2:2 * NBUF + 2 + NBUF]
    out_sem = scratch[2 * NBUF + 2 + NBUF:2 * NBUF + 2 + 2 * NBUF]

    wid = lax.axis_index("s") * NC + lax.axis_index("c")
    base_u = wid * UPW

    def unit_pos(g):
        u = base_u + g
        return u // BTILES, (u % BTILES) * 128

    def fire_in(g, b):
        ll, b0 = unit_pos(g)
        pltpu.async_copy(idx_hbm.at[ll, pl.ds(0, 2), pl.ds(b0, 128)],
                         idx_v[b], in_sem[b])

    def wait_in(b):
        pltpu.make_async_copy(idx_hbm.at[0, pl.ds(0, 2), pl.ds(0, 128)],
                              idx_v[b], in_sem[b]).wait()

    def fire_out(g, b):
        ll, b0 = unit_pos(g)
        pltpu.async_copy(
            dt_v[b], out_hbm.at[ll, pl.ds(DIM, DT), pl.ds(b0, 128)],
            out_sem[b])

    def wait_out(b):
        pltpu.make_async_copy(
            dt_v[b], out_hbm.at[0, pl.ds(DIM, DT), pl.ds(0, 128)],
            out_sem[b]).wait()

    pltpu.sync_copy(day_hbm, day_tab)
    pltpu.sync_copy(time_hbm, time_tab)

    for g0 in range(NBUF - 1):
        fire_in(g0, g0)

    @pl.loop(0, NOUTER)
    def _blk(k):
        for j in range(NBUF):
            g = k * NBUF + j
            b = j

            wait_in(b)

            f = g + NBUF - 1
            fb = (j + NBUF - 1) % NBUF

            @pl.when(f < UPW)
            def _():
                @pl.when(g >= 1)
                def _():
                    wait_out(fb)
                fire_in(f, fb)

            @pl.loop(0, 128 // LANES)
            def _q(q):
                d16 = idx_v[b][0, pl.ds(LANES * q, LANES)]
                t16 = idx_v[b][1, pl.ds(LANES * q, LANES)]
                dbase = DAY_SIZE * d16
                tbase = TIME_SIZE * t16
                for c in range(DAY_SIZE):
                    dt_v[b][c, pl.ds(LANES * q, LANES)] = plsc.load_gather(
                        day_tab, [dbase + c])
                for c in range(TIME_SIZE):
                    dt_v[b][DAY_SIZE + c, pl.ds(LANES * q, LANES)] = (
                        plsc.load_gather(time_tab, [tbase + c]))

            fire_out(g, b)

    for g in range(UPW - NBUF, UPW):
        wait_out(g % NBUF)


def _tc_body(inp_ref, prev_ref, out_ref):
    del prev_ref
    x = inp_ref[...]
    for l in range(8):
        out_ref[l] = jnp.transpose(x[:, l, :], (1, 0))


_tc_transpose = pl.pallas_call(
    _tc_body,
    out_shape=jax.ShapeDtypeStruct((L, OUT_D, B), jnp.float32),
    grid=(L // 8, BTILES),
    in_specs=[
        pl.BlockSpec((128, 8, DIM), lambda lt, bt: (bt, lt, 0)),
        pl.BlockSpec(memory_space=pl.ANY),
    ],
    out_specs=pl.BlockSpec((8, DIM, 128), lambda lt, bt: (lt, 0, bt)),
    input_output_aliases={1: 0},
)


def kernel(inp, daytime, emb_day, emb_time):
    idx_t = jnp.transpose(daytime.astype(jnp.int32), (1, 2, 0))
    part = _sc_body(idx_t,
                    emb_day.reshape(NUM_DAYS * DAY_SIZE),
                    emb_time.reshape(DAILY_TIMES * TIME_SIZE))
    full = _tc_transpose(inp, part)
    return jnp.transpose(full, (2, 0, 1))


# speedup vs baseline: 1.3256x; 1.3256x over previous
import functools

import jax
import jax.numpy as jnp
from jax import lax
from jax.experimental import pallas as pl
from jax.experimental.pallas import tpu as pltpu
from jax.experimental.pallas import tpu_sc as plsc

B, L, DIM = 1024, 200, 128
DAY_SIZE, TIME_SIZE = 32, 64
NUM_DAYS, DAILY_TIMES = 7, 288
DT = DAY_SIZE + TIME_SIZE
OUT_D = DIM + DT
N = B * L

_info = plsc.get_sparse_core_info()
NC, NS, LANES = _info.num_cores, _info.num_subcores, _info.num_lanes
NW = NC * NS
BTILES = B // 128
NUNIT = L * BTILES
UPW = NUNIT // NW
NBUF = 5
NOUTER = UPW // NBUF

_mesh = plsc.VectorSubcoreMesh(core_axis_name="c", subcore_axis_name="s")


@functools.partial(
    pl.kernel,
    out_type=jax.ShapeDtypeStruct((L, OUT_D, B), jnp.float32),
    mesh=_mesh,
    compiler_params=pltpu.CompilerParams(use_tc_tiling_on_sc=True,
                                         needs_layout_passes=False),
    scratch_types=(
        [pltpu.VMEM((2, 128), jnp.int32)] * NBUF
        + [pltpu.VMEM((DT, 128), jnp.float32)] * NBUF
        + [pltpu.VMEM((NUM_DAYS * DAY_SIZE,), jnp.float32)]
        + [pltpu.VMEM((DAILY_TIMES * TIME_SIZE,), jnp.float32)]
        + [pltpu.SemaphoreType.DMA] * (2 * NBUF)
    ),
)
def _sc_body(idx_hbm, day_hbm, time_hbm, out_hbm, *scratch):
    idx_v = scratch[0:NBUF]
    dt_v = scratch[NBUF:2 * NBUF]
    day_tab = scratch[2 * NBUF]
    time_tab = scratch[2 * NBUF + 1]
    in_sem = scratch[2 * NBUF + 2:2 * NBUF + 2 + NBUF]
    out_sem = scratch[2 * NBUF + 2 + NBUF:2 * NBUF + 2 + 2 * NBUF]

    wid = lax.axis_index("s") * NC + lax.axis_index("c")
    base_u = wid * UPW

    def unit_pos(g):
        u = base_u + g
        return u // BTILES, (u % BTILES) * 128

    def fire_in(g, b):
        ll, b0 = unit_pos(g)
        pltpu.async_copy(idx_hbm.at[ll, pl.ds(0, 2), pl.ds(b0, 128)],
                         idx_v[b], in_sem[b])

    def wait_in(b):
        pltpu.make_async_copy(idx_hbm.at[0, pl.ds(0, 2), pl.ds(0, 128)],
                              idx_v[b], in_sem[b]).wait()

    def fire_out(g, b):
        ll, b0 = unit_pos(g)
        pltpu.async_copy(
            dt_v[b], out_hbm.at[ll, pl.ds(DIM, DT), pl.ds(b0, 128)],
            out_sem[b])

    def wait_out(b):
        pltpu.make_async_copy(
            dt_v[b], out_hbm.at[0, pl.ds(DIM, DT), pl.ds(0, 128)],
            out_sem[b]).wait()

    pltpu.sync_copy(day_hbm, day_tab)
    pltpu.sync_copy(time_hbm, time_tab)

    for g0 in range(NBUF - 1):
        fire_in(g0, g0)

    @pl.loop(0, NOUTER)
    def _blk(k):
        for j in range(NBUF):
            g = k * NBUF + j
            b = j

            wait_in(b)

            f = g + NBUF - 1
            fb = (j + NBUF - 1) % NBUF

            @pl.when(f < UPW)
            def _():
                @pl.when(g >= 1)
                def _():
                    wait_out(fb)
                fire_in(f, fb)

            @pl.loop(0, 128 // LANES)
            def _q(q):
                d16 = idx_v[b][0, pl.ds(LANES * q, LANES)]
                t16 = idx_v[b][1, pl.ds(LANES * q, LANES)]
                dbase = DAY_SIZE * d16
                tbase = TIME_SIZE * t16
                for c0 in range(0, DAY_SIZE, 4):
                    vs = [plsc.load_gather(day_tab, [dbase + (c0 + i)])
                          for i in range(4)]
                    for i in range(4):
                        dt_v[b][c0 + i, pl.ds(LANES * q, LANES)] = vs[i]
                for c0 in range(0, TIME_SIZE, 4):
                    vs = [plsc.load_gather(time_tab, [tbase + (c0 + i)])
                          for i in range(4)]
                    for i in range(4):
                        dt_v[b][DAY_SIZE + c0 + i,
                                pl.ds(LANES * q, LANES)] = vs[i]

            fire_out(g, b)

    for g in range(UPW - NBUF, UPW):
        wait_out(g % NBUF)


def _tc_body(inp_ref, prev_ref, out_ref):
    del prev_ref
    x = inp_ref[...]
    for l in range(8):
        out_ref[l] = jnp.transpose(x[:, l, :], (1, 0))


_tc_transpose = pl.pallas_call(
    _tc_body,
    out_shape=jax.ShapeDtypeStruct((L, OUT_D, B), jnp.float32),
    grid=(L // 8, BTILES),
    in_specs=[
        pl.BlockSpec((128, 8, DIM), lambda lt, bt: (bt, lt, 0)),
        pl.BlockSpec(memory_space=pl.ANY),
    ],
    out_specs=pl.BlockSpec((8, DIM, 128), lambda lt, bt: (lt, 0, bt)),
    input_output_aliases={1: 0},
)


def kernel(inp, daytime, emb_day, emb_time):
    idx_t = jnp.transpose(daytime.astype(jnp.int32), (1, 2, 0))
    part = _sc_body(idx_t,
                    emb_day.reshape(NUM_DAYS * DAY_SIZE),
                    emb_time.reshape(DAILY_TIMES * TIME_SIZE))
    full = _tc_transpose(inp, part)
    return jnp.transpose(full, (2, 0, 1))


# speedup vs baseline: 1.5842x; 1.1951x over previous
import functools

import jax
import jax.numpy as jnp
from jax import lax
from jax.experimental import pallas as pl
from jax.experimental.pallas import tpu as pltpu
from jax.experimental.pallas import tpu_sc as plsc

B, L, DIM = 1024, 200, 128
DAY_SIZE, TIME_SIZE = 32, 64
NUM_DAYS, DAILY_TIMES = 7, 288
DT = DAY_SIZE + TIME_SIZE
OUT_D = DIM + DT
N = B * L

_info = plsc.get_sparse_core_info()
NC, NS, LANES = _info.num_cores, _info.num_subcores, _info.num_lanes
NW = NC * NS
BTILES = B // 128
NUNIT = L * BTILES
UPW = NUNIT // NW
NBUF = 5
NOUTER = UPW // NBUF

_mesh = plsc.VectorSubcoreMesh(core_axis_name="c", subcore_axis_name="s")


@functools.partial(
    pl.kernel,
    out_type=jax.ShapeDtypeStruct((L, OUT_D, B), jnp.float32),
    mesh=_mesh,
    compiler_params=pltpu.CompilerParams(use_tc_tiling_on_sc=True,
                                         needs_layout_passes=False),
    scratch_types=(
        [pltpu.VMEM((2, 128), jnp.int32)] * NBUF
        + [pltpu.VMEM((DT, 128), jnp.float32)] * NBUF
        + [pltpu.VMEM((NUM_DAYS * DAY_SIZE,), jnp.float32)]
        + [pltpu.VMEM((DAILY_TIMES * TIME_SIZE,), jnp.float32)]
        + [pltpu.SemaphoreType.DMA] * (2 * NBUF)
    ),
)
def _sc_body(idx_hbm, day_hbm, time_hbm, out_hbm, *scratch):
    idx_v = scratch[0:NBUF]
    dt_v = scratch[NBUF:2 * NBUF]
    day_tab = scratch[2 * NBUF]
    time_tab = scratch[2 * NBUF + 1]
    in_sem = scratch[2 * NBUF + 2:2 * NBUF + 2 + NBUF]
    out_sem = scratch[2 * NBUF + 2 + NBUF:2 * NBUF + 2 + 2 * NBUF]

    wid = lax.axis_index("s") * NC + lax.axis_index("c")
    base_u = wid * UPW

    def unit_pos(g):
        u = base_u + g
        return u // BTILES, (u % BTILES) * 128

    def fire_in(g, b):
        ll, b0 = unit_pos(g)
        pltpu.async_copy(idx_hbm.at[ll, pl.ds(0, 2), pl.ds(b0, 128)],
                         idx_v[b], in_sem[b])

    def wait_in(b):
        pltpu.make_async_copy(idx_hbm.at[0, pl.ds(0, 2), pl.ds(0, 128)],
                              idx_v[b], in_sem[b]).wait()

    def fire_out(g, b):
        ll, b0 = unit_pos(g)
        pltpu.async_copy(
            dt_v[b], out_hbm.at[ll, pl.ds(DIM, DT), pl.ds(b0, 128)],
            out_sem[b])

    def wait_out(b):
        pltpu.make_async_copy(
            dt_v[b], out_hbm.at[0, pl.ds(DIM, DT), pl.ds(0, 128)],
            out_sem[b]).wait()

    pltpu.sync_copy(day_hbm, day_tab)
    pltpu.sync_copy(time_hbm, time_tab)

    for g0 in range(NBUF - 1):
        fire_in(g0, g0)

    @pl.loop(0, NOUTER)
    def _blk(k):
        for j in range(NBUF):
            g = k * NBUF + j
            b = j

            wait_in(b)

            f = g + NBUF - 1
            fb = (j + NBUF - 1) % NBUF

            @pl.when(f < UPW)
            def _():
                @pl.when(g >= 1)
                def _():
                    wait_out(fb)
                fire_in(f, fb)

            @pl.loop(0, 128 // LANES)
            def _q(q):
                d16 = idx_v[b][0, pl.ds(LANES * q, LANES)]
                t16 = idx_v[b][1, pl.ds(LANES * q, LANES)]
                dbase = DAY_SIZE * d16
                tbase = TIME_SIZE * t16
                for c0 in range(0, DAY_SIZE, 4):
                    vs = [plsc.load_gather(day_tab, [dbase + (c0 + i)])
                          for i in range(4)]
                    for i in range(4):
                        dt_v[b][c0 + i, pl.ds(LANES * q, LANES)] = vs[i]
                for c0 in range(0, TIME_SIZE, 4):
                    vs = [plsc.load_gather(time_tab, [tbase + (c0 + i)])
                          for i in range(4)]
                    for i in range(4):
                        dt_v[b][DAY_SIZE + c0 + i,
                                pl.ds(LANES * q, LANES)] = vs[i]

            fire_out(g, b)

    for g in range(UPW - NBUF, UPW):
        wait_out(g % NBUF)


_LBLK = 40


def _tc_body(inp_ref, prev_ref, out_ref):
    del prev_ref
    x = inp_ref[...]
    for l in range(_LBLK):
        out_ref[l] = jnp.transpose(x[:, l, :], (1, 0))


_tc_transpose = pl.pallas_call(
    _tc_body,
    out_shape=jax.ShapeDtypeStruct((L, OUT_D, B), jnp.float32),
    grid=(L // _LBLK, BTILES),
    in_specs=[
        pl.BlockSpec((128, _LBLK, DIM), lambda lt, bt: (bt, lt, 0)),
        pl.BlockSpec(memory_space=pl.ANY),
    ],
    out_specs=pl.BlockSpec((_LBLK, DIM, 128), lambda lt, bt: (lt, 0, bt)),
    input_output_aliases={1: 0},
)


def kernel(inp, daytime, emb_day, emb_time):
    idx_t = jnp.transpose(daytime.astype(jnp.int32), (1, 2, 0))
    part = _sc_body(idx_t,
                    emb_day.reshape(NUM_DAYS * DAY_SIZE),
                    emb_time.reshape(DAILY_TIMES * TIME_SIZE))
    full = _tc_transpose(inp, part)
    return jnp.transpose(full, (2, 0, 1))


# speedup vs baseline: 1.6834x; 1.0626x over previous
import functools

import jax
import jax.numpy as jnp
from jax import lax
from jax.experimental import pallas as pl
from jax.experimental.pallas import tpu as pltpu
from jax.experimental.pallas import tpu_sc as plsc

B, L, DIM = 1024, 200, 128
DAY_SIZE, TIME_SIZE = 32, 64
NUM_DAYS, DAILY_TIMES = 7, 288
DT = DAY_SIZE + TIME_SIZE
OUT_D = DIM + DT
N = B * L

_info = plsc.get_sparse_core_info()
NC, NS, LANES = _info.num_cores, _info.num_subcores, _info.num_lanes
NW = NC * NS
BTILES = B // 128
NUNIT = L * BTILES
UPW = NUNIT // NW
NBUF = 5
NOUTER = UPW // NBUF

_mesh = plsc.VectorSubcoreMesh(core_axis_name="c", subcore_axis_name="s")


@functools.partial(
    pl.kernel,
    out_type=jax.ShapeDtypeStruct((L, OUT_D, B), jnp.float32),
    mesh=_mesh,
    compiler_params=pltpu.CompilerParams(use_tc_tiling_on_sc=True,
                                         needs_layout_passes=False),
    scratch_types=(
        [pltpu.VMEM((2, 128), jnp.int32)] * NBUF
        + [pltpu.VMEM((DT, 128), jnp.float32)] * NBUF
        + [pltpu.VMEM((NUM_DAYS * DAY_SIZE,), jnp.float32)]
        + [pltpu.VMEM((DAILY_TIMES * TIME_SIZE,), jnp.float32)]
        + [pltpu.SemaphoreType.DMA] * (2 * NBUF)
    ),
)
def _sc_body(idx_hbm, day_hbm, time_hbm, out_hbm, *scratch):
    idx_v = scratch[0:NBUF]
    dt_v = scratch[NBUF:2 * NBUF]
    day_tab = scratch[2 * NBUF]
    time_tab = scratch[2 * NBUF + 1]
    in_sem = scratch[2 * NBUF + 2:2 * NBUF + 2 + NBUF]
    out_sem = scratch[2 * NBUF + 2 + NBUF:2 * NBUF + 2 + 2 * NBUF]

    wid = lax.axis_index("s") * NC + lax.axis_index("c")
    base_u = wid * UPW

    def unit_pos(g):
        u = base_u + g
        return u // BTILES, (u % BTILES) * 128

    def fire_in(g, b):
        ll, b0 = unit_pos(g)
        pltpu.async_copy(idx_hbm.at[ll, pl.ds(0, 2), pl.ds(b0, 128)],
                         idx_v[b], in_sem[b])

    def wait_in(b):
        pltpu.make_async_copy(idx_hbm.at[0, pl.ds(0, 2), pl.ds(0, 128)],
                              idx_v[b], in_sem[b]).wait()

    def fire_out(g, b):
        ll, b0 = unit_pos(g)
        pltpu.async_copy(
            dt_v[b], out_hbm.at[ll, pl.ds(DIM, DT), pl.ds(b0, 128)],
            out_sem[b])

    def wait_out(b):
        pltpu.make_async_copy(
            dt_v[b], out_hbm.at[0, pl.ds(DIM, DT), pl.ds(0, 128)],
            out_sem[b]).wait()

    pltpu.sync_copy(day_hbm, day_tab)
    pltpu.sync_copy(time_hbm, time_tab)

    for g0 in range(NBUF - 1):
        fire_in(g0, g0)

    @pl.loop(0, NOUTER)
    def _blk(k):
        for j in range(NBUF):
            g = k * NBUF + j
            b = j

            wait_in(b)

            f = g + NBUF - 1
            fb = (j + NBUF - 1) % NBUF

            @pl.when(f < UPW)
            def _():
                @pl.when(g >= 1)
                def _():
                    wait_out(fb)
                fire_in(f, fb)

            @pl.loop(0, 128 // LANES, unroll=2)
            def _q(q):
                d16 = idx_v[b][0, pl.ds(LANES * q, LANES)]
                t16 = idx_v[b][1, pl.ds(LANES * q, LANES)]
                dbase = DAY_SIZE * d16
                tbase = TIME_SIZE * t16
                for c0 in range(0, DAY_SIZE, 8):
                    vs = [plsc.load_gather(day_tab, [dbase + (c0 + i)])
                          for i in range(8)]
                    for i in range(8):
                        dt_v[b][c0 + i, pl.ds(LANES * q, LANES)] = vs[i]
                for c0 in range(0, TIME_SIZE, 8):
                    vs = [plsc.load_gather(time_tab, [tbase + (c0 + i)])
                          for i in range(8)]
                    for i in range(8):
                        dt_v[b][DAY_SIZE + c0 + i,
                                pl.ds(LANES * q, LANES)] = vs[i]

            fire_out(g, b)

    for g in range(UPW - NBUF, UPW):
        wait_out(g % NBUF)


_LBLK = 40


def _tc_body(inp_ref, prev_ref, out_ref):
    del prev_ref
    x = inp_ref[...]
    for l in range(_LBLK):
        out_ref[l] = jnp.transpose(x[:, l, :], (1, 0))


_tc_transpose = pl.pallas_call(
    _tc_body,
    out_shape=jax.ShapeDtypeStruct((L, OUT_D, B), jnp.float32),
    grid=(L // _LBLK, BTILES),
    in_specs=[
        pl.BlockSpec((128, _LBLK, DIM), lambda lt, bt: (bt, lt, 0)),
        pl.BlockSpec(memory_space=pl.ANY),
    ],
    out_specs=pl.BlockSpec((_LBLK, DIM, 128), lambda lt, bt: (lt, 0, bt)),
    input_output_aliases={1: 0},
)


def kernel(inp, daytime, emb_day, emb_time):
    idx_t = jnp.transpose(daytime.astype(jnp.int32), (1, 2, 0))
    part = _sc_body(idx_t,
                    emb_day.reshape(NUM_DAYS * DAY_SIZE),
                    emb_time.reshape(DAILY_TIMES * TIME_SIZE))
    full = _tc_transpose(inp, part)
    return jnp.transpose(full, (2, 0, 1))
